# baseline (speedup 1.0000x reference)
import functools

import jax
import jax.numpy as jnp
from jax import lax
from jax.experimental import pallas as pl
from jax.experimental.pallas import tpu as pltpu

_B = 8
_A = 5
_NATT = 85
_HW = 1024
_N = _A * _HW
_G = _N // 128
_K = 1000
_KPAD = 1024
_M = 1280
_STRIDE = 32.0
_NMS_THR = 0.5
_ONE_BITS = 0x3F800000


def _sig(x):
    return jax.nn.sigmoid(x)


def _dot_nt(a, b):
    return lax.dot_general(a, b, (((1,), (1,)), ((), ())),
                           preferred_element_type=jnp.float32,
                           precision=lax.Precision.HIGHEST)


def _dot_nn(a, b, precision=None):
    return lax.dot_general(a, b, (((1,), (0,)), ((), ())),
                           preferred_element_type=jnp.float32,
                           precision=precision)


def _yolof_body(pred_ref, anch_ref, out_ref,
                d3_ref, pos_ref, c_ref, ct_ref, sl_ref, ss_ref):
    for a in range(_A):
        pa = pred_ref[0, a * _NATT:(a + 1) * _NATT, :, :]
        an = anch_ref[a * 4:(a + 1) * 4, :, :]
        cx = _sig(pa[0]) * _STRIDE + an[0]
        cy = _sig(pa[1]) * _STRIDE + an[1]
        w = jnp.exp(jnp.clip(pa[2], -10.0, 8.0)) * an[2]
        h = jnp.exp(jnp.clip(pa[3], -10.0, 8.0)) * an[3]
        cls_max = jnp.max(pa[5:_NATT], axis=0)
        score = _sig(pa[4]) * _sig(cls_max)
        gs = pl.ds(a * 8, 8)
        d3_ref[0, gs, :] = cx - w * 0.5
        d3_ref[1, gs, :] = cy - h * 0.5
        d3_ref[2, gs, :] = cx + w * 0.5
        d3_ref[3, gs, :] = cy + h * 0.5
        d3_ref[4, gs, :] = score
    idx = (lax.broadcasted_iota(jnp.int32, (_G, 128), 0) * 128
           + lax.broadcasted_iota(jnp.int32, (_G, 128), 1))
    d3_ref[5, :, :] = idx.astype(jnp.float32)
    d3_ref[6:8, :, :] = jnp.zeros((2, _G, 128), jnp.float32)

    bits = lax.bitcast_convert_type(d3_ref[4, :, :], jnp.int32)

    def bis_step(_, lh):
        lo, hi = lh
        mid = (lo + hi) // 2
        c = jnp.sum((bits > mid).astype(jnp.int32))
        takes = c < _KPAD
        return (jnp.where(takes, lo, mid + 1), jnp.where(takes, mid, hi))

    lo, _hi = lax.fori_loop(0, 31, bis_step,
                            (jnp.int32(0), jnp.int32(_ONE_BITS)))
    sel = (bits >= lo)
    self_f = sel.astype(jnp.float32)

    su = (lax.broadcasted_iota(jnp.int32, (128, 128), 0)
          < lax.broadcasted_iota(jnp.int32, (128, 128), 1)).astype(jnp.float32)
    within = _dot_nn(self_f, su)
    grptot = within[:, 127:128] + self_f[:, 127:128]
    l40 = (lax.broadcasted_iota(jnp.int32, (_G, _G), 1)
           < lax.broadcasted_iota(jnp.int32, (_G, _G), 0)).astype(jnp.float32)
    groffs = _dot_nn(l40, grptot)
    posf = groffs + within
    pos_ref[...] = jnp.where(sel, posf.astype(jnp.int32), -1)

    miota = lax.broadcasted_iota(jnp.int32, (_M, 1), 0)

    def comp_step(g, cct):
        c_acc, ct_acc = cct
        pg = pos_ref[pl.ds(g, 1), :]
        oh = (pg == miota).astype(jnp.float32)
        dg = d3_ref[:, pl.ds(g, 1), :].reshape(8, 128)
        return (c_acc + _dot_nt(oh, dg), ct_acc + _dot_nt(dg, oh))

    c_val, ct_val = lax.fori_loop(
        0, _G, comp_step,
        (jnp.zeros((_M, 8), jnp.float32), jnp.zeros((8, _M), jnp.float32)))
    c_ref[...] = c_val
    ct_ref[...] = ct_val

    s_lane = ct_ref[4:5, :]
    i_lane = ct_ref[5:6, :]

    def rank_step(jc, rank):
        rs = pl.ds(jc * 128, 128)
        sj = c_ref[rs, 4:5]
        ij = c_ref[rs, 5:6]
        gt = (sj > s_lane) | ((sj == s_lane) & (ij < i_lane))
        return rank + jnp.sum(gt.astype(jnp.int32), axis=0, keepdims=True)

    rank = lax.fori_loop(0, _M // 128, rank_step,
                         jnp.zeros((1, _M), jnp.int32))

    c_full = c_ref[...]
    ct_full = ct_ref[...]

    def gather_step(kc, _):
        kk = kc * 128 + lax.broadcasted_iota(jnp.int32, (128, 1), 0)
        oh2 = (rank == kk).astype(jnp.float32)
        sl_ref[:, pl.ds(kc * 128, 128)] = _dot_nt(ct_full, oh2)
        ss_ref[pl.ds(kc * 128, 128), :] = _dot_nn(
            oh2, c_full, precision=lax.Precision.HIGHEST)
        return 0

    lax.fori_loop(0, _KPAD // 128, gather_step, 0)

    xj1 = sl_ref[0:1, :]
    yj1 = sl_ref[1:2, :]
    xj2 = sl_ref[2:3, :]
    yj2 = sl_ref[3:4, :]
    aj = (xj2 - xj1) * (yj2 - yj1)
    colid = lax.broadcasted_iota(jnp.int32, (1, _KPAD), 1)

    def nms_step(ic, sup):
        rs = pl.ds(ic * 128, 128)
        xi1 = ss_ref[rs, 0:1]
        yi1 = ss_ref[rs, 1:2]
        xi2 = ss_ref[rs, 2:3]
        yi2 = ss_ref[rs, 3:4]
        ix = jnp.clip(jnp.minimum(xi2, xj2) - jnp.maximum(xi1, xj1), 0.0, None)
        iy = jnp.clip(jnp.minimum(yi2, yj2) - jnp.maximum(yi1, yj1), 0.0, None)
        inter = ix * iy
        ai = (xi2 - xi1) * (yi2 - yi1)
        uni = jnp.maximum(ai + aj - inter, 1e-6)
        rowid = ic * 128 + lax.broadcasted_iota(jnp.int32, (128, 1), 0)
        bad = (inter > _NMS_THR * uni) & (rowid < colid)
        badf = jnp.where(bad, 1.0, 0.0)
        return jnp.maximum(sup, jnp.max(badf, axis=0, keepdims=True))

    sup = lax.fori_loop(0, _KPAD // 128, nms_step,
                        jnp.zeros((1, _KPAD), jnp.float32))
    keep = 1.0 - sup

    out_ref[0, 0:4, :] = sl_ref[0:4, :]
    out_ref[0, 4:5, :] = sl_ref[4:5, :] * keep
    out_ref[0, 5:8, :] = jnp.zeros((3, _KPAD), jnp.float32)


@jax.jit
def kernel(pred_map, anchors):
    pm = pred_map.reshape(_B, _A * _NATT, 8, 128)
    an = (anchors.reshape(_A, 8, 128, 4).transpose(0, 3, 1, 2)
          .reshape(_A * 4, 8, 128))
    out = pl.pallas_call(
        _yolof_body,
        grid=(_B,),
        in_specs=[
            pl.BlockSpec((1, _A * _NATT, 8, 128), lambda b: (b, 0, 0, 0)),
            pl.BlockSpec((_A * 4, 8, 128), lambda b: (0, 0, 0)),
        ],
        out_specs=pl.BlockSpec((1, 8, _KPAD), lambda b: (b, 0, 0)),
        out_shape=jax.ShapeDtypeStruct((_B, 8, _KPAD), jnp.float32),
        scratch_shapes=[
            pltpu.VMEM((8, _G, 128), jnp.float32),
            pltpu.VMEM((_G, 128), jnp.int32),
            pltpu.VMEM((_M, 8), jnp.float32),
            pltpu.VMEM((8, _M), jnp.float32),
            pltpu.VMEM((8, _KPAD), jnp.float32),
            pltpu.VMEM((_KPAD, 8), jnp.float32),
        ],
    )(pm, an)
    return jnp.transpose(out, (0, 2, 1))[:, :_K, :5]


# speedup vs baseline: 2.6444x; 2.6444x over previous
import functools

import jax
import jax.numpy as jnp
from jax import lax
from jax.experimental import pallas as pl
from jax.experimental.pallas import tpu as pltpu

_B = 8
_A = 5
_NATT = 85
_HW = 1024
_N = _A * _HW
_G = _N // 128
_K = 1000
_KPAD = 1024
_M = 1280
_STRIDE = 32.0
_NMS_THR = 0.5
_ONE_BITS = 0x3F800000


def _sig(x):
    return jax.nn.sigmoid(x)


def _dot_nt(a, b):
    return lax.dot_general(a, b, (((1,), (1,)), ((), ())),
                           preferred_element_type=jnp.float32,
                           precision=lax.Precision.HIGHEST)


def _dot_nn(a, b, precision=None):
    return lax.dot_general(a, b, (((1,), (0,)), ((), ())),
                           preferred_element_type=jnp.float32,
                           precision=precision)


def _yolof_body(pred_ref, anch_ref, out_ref,
                d3_ref, pos_ref, c_ref, ct_ref, sl_ref, ss_ref):
    for a in range(_A):
        pa = pred_ref[0, a * _NATT:(a + 1) * _NATT, :, :]
        an = anch_ref[a * 4:(a + 1) * 4, :, :]
        cx = _sig(pa[0]) * _STRIDE + an[0]
        cy = _sig(pa[1]) * _STRIDE + an[1]
        w = jnp.exp(jnp.clip(pa[2], -10.0, 8.0)) * an[2]
        h = jnp.exp(jnp.clip(pa[3], -10.0, 8.0)) * an[3]
        cls_max = jnp.max(pa[5:_NATT], axis=0)
        score = _sig(pa[4]) * _sig(cls_max)
        gs = pl.ds(a * 8, 8)
        d3_ref[0, gs, :] = cx - w * 0.5
        d3_ref[1, gs, :] = cy - h * 0.5
        d3_ref[2, gs, :] = cx + w * 0.5
        d3_ref[3, gs, :] = cy + h * 0.5
        d3_ref[4, gs, :] = score
    idx = (lax.broadcasted_iota(jnp.int32, (_G, 128), 0) * 128
           + lax.broadcasted_iota(jnp.int32, (_G, 128), 1))
    d3_ref[5, :, :] = idx.astype(jnp.float32)
    d3_ref[6:8, :, :] = jnp.zeros((2, _G, 128), jnp.float32)

    bits = lax.bitcast_convert_type(d3_ref[4, :, :], jnp.int32)

    def bis_step(_, lh):
        lo, hi = lh
        mid = (lo + hi) // 2
        c = jnp.sum((bits > mid).astype(jnp.int32))
        takes = c < _KPAD
        return (jnp.where(takes, lo, mid + 1), jnp.where(takes, mid, hi))

    lo, _hi = lax.fori_loop(0, 31, bis_step,
                            (jnp.int32(0), jnp.int32(_ONE_BITS)))
    sel = (bits >= lo)
    self_f = sel.astype(jnp.float32)

    su = (lax.broadcasted_iota(jnp.int32, (128, 128), 0)
          < lax.broadcasted_iota(jnp.int32, (128, 128), 1)).astype(jnp.float32)
    within = _dot_nn(self_f, su)
    pos_ref[...] = jnp.where(sel, within.astype(jnp.int32), -1)

    c_ref[...] = jnp.zeros((_M, 8), jnp.float32)
    miota = lax.broadcasted_iota(jnp.int32, (128, 1), 0)

    def comp_step(g, off):
        pg = pos_ref[pl.ds(g, 1), :]
        cnt = jnp.sum((pg >= 0).astype(jnp.int32))
        oh = (pg == miota).astype(jnp.float32)
        dg = d3_ref[:, pl.ds(g, 1), :].reshape(8, 128)
        offc = jnp.minimum(off, _M - 128)
        rw = pl.ds(offc, 128)
        c_ref[rw, :] = c_ref[rw, :] + _dot_nt(oh, dg)
        return off + cnt

    lax.fori_loop(0, _G, comp_step, jnp.int32(0))
    for k in range(_M // 128):
        ct_ref[:, k * 128:(k + 1) * 128] = jnp.transpose(
            c_ref[k * 128:(k + 1) * 128, :])

    s_lane = ct_ref[4:5, :]
    i_lane = ct_ref[5:6, :]

    def rank_step(jc, rank):
        rs = pl.ds(jc * 128, 128)
        sj = c_ref[rs, 4:5]
        ij = c_ref[rs, 5:6]
        gt = (sj > s_lane) | ((sj == s_lane) & (ij < i_lane))
        return rank + jnp.sum(gt.astype(jnp.int32), axis=0, keepdims=True)

    rank = lax.fori_loop(0, _M // 128, rank_step,
                         jnp.zeros((1, _M), jnp.int32))

    ct_full = ct_ref[...]

    def gather_step(kc, _):
        kk = kc * 128 + lax.broadcasted_iota(jnp.int32, (128, 1), 0)
        oh2 = (rank == kk).astype(jnp.float32)
        lane = _dot_nt(ct_full, oh2)
        sl_ref[:, pl.ds(kc * 128, 128)] = lane
        ss_ref[pl.ds(kc * 128, 128), :] = jnp.transpose(lane)
        return 0

    lax.fori_loop(0, _KPAD // 128, gather_step, 0)

    xj1 = sl_ref[0:1, :]
    yj1 = sl_ref[1:2, :]
    xj2 = sl_ref[2:3, :]
    yj2 = sl_ref[3:4, :]
    aj = (xj2 - xj1) * (yj2 - yj1)
    colid = lax.broadcasted_iota(jnp.int32, (1, _KPAD), 1)

    def nms_step(ic, sup):
        rs = pl.ds(ic * 128, 128)
        xi1 = ss_ref[rs, 0:1]
        yi1 = ss_ref[rs, 1:2]
        xi2 = ss_ref[rs, 2:3]
        yi2 = ss_ref[rs, 3:4]
        ix = jnp.clip(jnp.minimum(xi2, xj2) - jnp.maximum(xi1, xj1), 0.0, None)
        iy = jnp.clip(jnp.minimum(yi2, yj2) - jnp.maximum(yi1, yj1), 0.0, None)
        inter = ix * iy
        ai = (xi2 - xi1) * (yi2 - yi1)
        uni = jnp.maximum(ai + aj - inter, 1e-6)
        rowid = ic * 128 + lax.broadcasted_iota(jnp.int32, (128, 1), 0)
        bad = (inter > _NMS_THR * uni) & (rowid < colid)
        badf = jnp.where(bad, 1.0, 0.0)
        return jnp.maximum(sup, jnp.max(badf, axis=0, keepdims=True))

    sup = lax.fori_loop(0, _KPAD // 128, nms_step,
                        jnp.zeros((1, _KPAD), jnp.float32))
    keep = 1.0 - sup

    out_ref[0, 0:4, :] = sl_ref[0:4, :]
    out_ref[0, 4:5, :] = sl_ref[4:5, :] * keep
    out_ref[0, 5:8, :] = jnp.zeros((3, _KPAD), jnp.float32)


@jax.jit
def kernel(pred_map, anchors):
    pm = pred_map.reshape(_B, _A * _NATT, 8, 128)
    an = (anchors.reshape(_A, 8, 128, 4).transpose(0, 3, 1, 2)
          .reshape(_A * 4, 8, 128))
    out = pl.pallas_call(
        _yolof_body,
        grid=(_B,),
        in_specs=[
            pl.BlockSpec((1, _A * _NATT, 8, 128), lambda b: (b, 0, 0, 0)),
            pl.BlockSpec((_A * 4, 8, 128), lambda b: (0, 0, 0)),
        ],
        out_specs=pl.BlockSpec((1, 8, _KPAD), lambda b: (b, 0, 0)),
        out_shape=jax.ShapeDtypeStruct((_B, 8, _KPAD), jnp.float32),
        scratch_shapes=[
            pltpu.VMEM((8, _G, 128), jnp.float32),
            pltpu.VMEM((_G, 128), jnp.int32),
            pltpu.VMEM((_M, 8), jnp.float32),
            pltpu.VMEM((8, _M), jnp.float32),
            pltpu.VMEM((8, _KPAD), jnp.float32),
            pltpu.VMEM((_KPAD, 8), jnp.float32),
        ],
    )(pm, an)
    return jnp.transpose(out, (0, 2, 1))[:, :_K, :5]


# speedup vs baseline: 3.3996x; 1.2856x over previous
import functools

import jax
import jax.numpy as jnp
from jax import lax
from jax.experimental import pallas as pl
from jax.experimental.pallas import tpu as pltpu

_B = 8
_A = 5
_NATT = 85
_HW = 1024
_N = _A * _HW
_G = _N // 128
_K = 1000
_KPAD = 1024
_M = 1280
_STRIDE = 32.0
_NMS_THR = 0.5
_ONE_BITS = 0x3F800000


def _sig(x):
    return jax.nn.sigmoid(x)


def _dot_nt(a, b):
    return lax.dot_general(a, b, (((1,), (1,)), ((), ())),
                           preferred_element_type=jnp.float32,
                           precision=lax.Precision.HIGHEST)


def _dot_nn(a, b, precision=None):
    return lax.dot_general(a, b, (((1,), (0,)), ((), ())),
                           preferred_element_type=jnp.float32,
                           precision=precision)


def _yolof_body(pred_ref, anch_ref, out_ref,
                d3_ref, pos_ref, c_ref, ct_ref, sl_ref, ss_ref):
    for a in range(_A):
        pa = pred_ref[0, a * _NATT:(a + 1) * _NATT, :, :]
        an = anch_ref[a * 4:(a + 1) * 4, :, :]
        cx = _sig(pa[0]) * _STRIDE + an[0]
        cy = _sig(pa[1]) * _STRIDE + an[1]
        w = jnp.exp(jnp.clip(pa[2], -10.0, 8.0)) * an[2]
        h = jnp.exp(jnp.clip(pa[3], -10.0, 8.0)) * an[3]
        cls_max = jnp.max(pa[5:_NATT], axis=0)
        score = _sig(pa[4]) * _sig(cls_max)
        gs = pl.ds(a * 8, 8)
        d3_ref[0, gs, :] = cx - w * 0.5
        d3_ref[1, gs, :] = cy - h * 0.5
        d3_ref[2, gs, :] = cx + w * 0.5
        d3_ref[3, gs, :] = cy + h * 0.5
        d3_ref[4, gs, :] = score
    d3_ref[5:8, :, :] = jnp.zeros((3, _G, 128), jnp.float32)

    bits = lax.bitcast_convert_type(d3_ref[4, :, :], jnp.int32)

    def bis_step(_, lh):
        lo, hi = lh
        span = hi - lo
        m1 = lo + span // 4
        m2 = lo + span // 2
        m3 = lo + (span - span // 4)
        p1 = jnp.sum((bits > m1).astype(jnp.int32)) < _KPAD
        p2 = jnp.sum((bits > m2).astype(jnp.int32)) < _KPAD
        p3 = jnp.sum((bits > m3).astype(jnp.int32)) < _KPAD
        nlo = jnp.where(p1, lo, jnp.where(p2, m1 + 1,
                                          jnp.where(p3, m2 + 1, m3 + 1)))
        nhi = jnp.where(p1, m1, jnp.where(p2, m2, jnp.where(p3, m3, hi)))
        return (nlo, nhi)

    lo, _hi = lax.fori_loop(0, 16, bis_step,
                            (jnp.int32(0), jnp.int32(_ONE_BITS)))
    sel = (bits >= lo)
    self_f = sel.astype(jnp.float32)

    su = (lax.broadcasted_iota(jnp.int32, (128, 128), 0)
          < lax.broadcasted_iota(jnp.int32, (128, 128), 1)).astype(jnp.float32)
    within = _dot_nn(self_f, su)
    pos_ref[...] = jnp.where(sel, within.astype(jnp.int32), -1)

    c_ref[...] = jnp.zeros((_M, 8), jnp.float32)
    miota = lax.broadcasted_iota(jnp.int32, (128, 1), 0)

    def comp_step(g, off):
        pg = pos_ref[pl.ds(g, 1), :]
        cnt = jnp.sum((pg >= 0).astype(jnp.int32))
        oh = (pg == miota).astype(jnp.float32)
        dg = d3_ref[:, pl.ds(g, 1), :].reshape(8, 128)
        offc = jnp.minimum(off, _M - 128)
        rw = pl.ds(offc, 128)
        c_ref[rw, :] = c_ref[rw, :] + _dot_nt(oh, dg)
        return off + cnt

    lax.fori_loop(0, _G, comp_step, jnp.int32(0))
    for k in range(_M // 128):
        ct_ref[:, k * 128:(k + 1) * 128] = jnp.transpose(
            c_ref[k * 128:(k + 1) * 128, :])

    s_lane = ct_ref[4:5, :]
    m_lane = lax.broadcasted_iota(jnp.int32, (1, _M), 1)

    def rank_step(jc, rank):
        rs = pl.ds(jc * 128, 128)
        sj = c_ref[rs, 4:5]
        mj = jc * 128 + lax.broadcasted_iota(jnp.int32, (128, 1), 0)
        gt = (sj > s_lane) | ((sj == s_lane) & (mj < m_lane))
        return rank + jnp.sum(gt.astype(jnp.int32), axis=0, keepdims=True)

    rank = lax.fori_loop(0, _M // 128, rank_step,
                         jnp.zeros((1, _M), jnp.int32))

    ct_full = ct_ref[...]

    def gather_step(kc, _):
        kk = kc * 128 + lax.broadcasted_iota(jnp.int32, (128, 1), 0)
        oh2 = (rank == kk).astype(jnp.float32)
        lane = lax.dot_general(ct_full, oh2, (((1,), (1,)), ((), ())),
                               preferred_element_type=jnp.float32)
        sl_ref[:, pl.ds(kc * 128, 128)] = lane
        ss_ref[pl.ds(kc * 128, 128), :] = jnp.transpose(lane)
        return 0

    lax.fori_loop(0, _KPAD // 128, gather_step, 0)

    xj1 = sl_ref[0:1, :]
    yj1 = sl_ref[1:2, :]
    xj2 = sl_ref[2:3, :]
    yj2 = sl_ref[3:4, :]
    aj = (xj2 - xj1) * (yj2 - yj1)

    sup = jnp.zeros((1, _KPAD), jnp.float32)
    for ic in range(_KPAD // 128):
        base = ic * 128
        w = _KPAD - base
        rs = pl.ds(base, 128)
        xi1 = ss_ref[rs, 0:1]
        yi1 = ss_ref[rs, 1:2]
        xi2 = ss_ref[rs, 2:3]
        yi2 = ss_ref[rs, 3:4]
        cxj1 = xj1[:, base:]
        cyj1 = yj1[:, base:]
        cxj2 = xj2[:, base:]
        cyj2 = yj2[:, base:]
        ix = jnp.clip(jnp.minimum(xi2, cxj2) - jnp.maximum(xi1, cxj1),
                      0.0, None)
        iy = jnp.clip(jnp.minimum(yi2, cyj2) - jnp.maximum(yi1, cyj1),
                      0.0, None)
        inter = ix * iy
        ai = (xi2 - xi1) * (yi2 - yi1)
        uni = jnp.maximum(ai + aj[:, base:] - inter, 1e-6)
        rowid = base + lax.broadcasted_iota(jnp.int32, (128, 1), 0)
        colg = base + lax.broadcasted_iota(jnp.int32, (1, w), 1)
        bad = (inter > _NMS_THR * uni) & (rowid < colg)
        colpart = jnp.max(jnp.where(bad, 1.0, 0.0), axis=0, keepdims=True)
        if base:
            colpart = jnp.concatenate(
                [jnp.zeros((1, base), jnp.float32), colpart], axis=1)
        sup = jnp.maximum(sup, colpart)
    keep = 1.0 - sup

    out_ref[0, 0:4, :] = sl_ref[0:4, :]
    out_ref[0, 4:5, :] = sl_ref[4:5, :] * keep
    out_ref[0, 5:8, :] = jnp.zeros((3, _KPAD), jnp.float32)


@jax.jit
def kernel(pred_map, anchors):
    pm = pred_map.reshape(_B, _A * _NATT, 8, 128)
    an = (anchors.reshape(_A, 8, 128, 4).transpose(0, 3, 1, 2)
          .reshape(_A * 4, 8, 128))
    out = pl.pallas_call(
        _yolof_body,
        grid=(_B,),
        in_specs=[
            pl.BlockSpec((1, _A * _NATT, 8, 128), lambda b: (b, 0, 0, 0)),
            pl.BlockSpec((_A * 4, 8, 128), lambda b: (0, 0, 0)),
        ],
        out_specs=pl.BlockSpec((1, 8, _KPAD), lambda b: (b, 0, 0)),
        out_shape=jax.ShapeDtypeStruct((_B, 8, _KPAD), jnp.float32),
        scratch_shapes=[
            pltpu.VMEM((8, _G, 128), jnp.float32),
            pltpu.VMEM((_G, 128), jnp.int32),
            pltpu.VMEM((_M, 8), jnp.float32),
            pltpu.VMEM((8, _M), jnp.float32),
            pltpu.VMEM((8, _KPAD), jnp.float32),
            pltpu.VMEM((_KPAD, 8), jnp.float32),
        ],
    )(pm, an)
    return jnp.transpose(out, (0, 2, 1))[:, :_K, :5]


# speedup vs baseline: 5.7151x; 1.6811x over previous
import functools

import jax
import jax.numpy as jnp
from jax import lax
from jax.experimental import pallas as pl
from jax.experimental.pallas import tpu as pltpu

_B = 8
_A = 5
_NATT = 85
_HW = 1024
_N = _A * _HW
_G = _N // 128
_K = 1000
_KPAD = 1024
_M = 1152
_STRIDE = 32.0
_NMS_THR = 0.5
_ONE_BITS = 0x3F800000


def _sig(x):
    return jax.nn.sigmoid(x)


def _dot_nt(a, b):
    return lax.dot_general(a, b, (((1,), (1,)), ((), ())),
                           preferred_element_type=jnp.float32)


def _dot_nn(a, b):
    return lax.dot_general(a, b, (((1,), (0,)), ((), ())),
                           preferred_element_type=jnp.float32)


def _yolof_body(pred_ref, anch_ref, out_ref,
                d3_ref, pos_ref, offs_ref, c_ref, ct_ref, sl_ref, ss_ref):
    for a in range(_A):
        pa = pred_ref[0, a * _NATT:(a + 1) * _NATT, :, :]
        an = anch_ref[a * 4:(a + 1) * 4, :, :]
        cx = _sig(pa[0]) * _STRIDE + an[0]
        cy = _sig(pa[1]) * _STRIDE + an[1]
        w = jnp.exp(jnp.clip(pa[2], -10.0, 8.0)) * an[2]
        h = jnp.exp(jnp.clip(pa[3], -10.0, 8.0)) * an[3]
        cls_max = jnp.max(pa[5:_NATT], axis=0)
        score = _sig(pa[4]) * _sig(cls_max)
        gs = pl.ds(a * 8, 8)
        d3_ref[gs, 0, :] = cx - w * 0.5
        d3_ref[gs, 1, :] = cy - h * 0.5
        d3_ref[gs, 2, :] = cx + w * 0.5
        d3_ref[gs, 3, :] = cy + h * 0.5
        d3_ref[gs, 4, :] = score
    d3_ref[:, 5:8, :] = jnp.zeros((_G, 3, 128), jnp.float32)

    bits = lax.bitcast_convert_type(d3_ref[:, 4, :], jnp.int32)

    def bis_step(_, lh):
        lo, hi = lh
        span = hi - lo
        m1 = lo + span // 4
        m2 = lo + span // 2
        m3 = lo + (span - span // 4)
        p1 = jnp.sum((bits > m1).astype(jnp.int32)) < _KPAD
        p2 = jnp.sum((bits > m2).astype(jnp.int32)) < _KPAD
        p3 = jnp.sum((bits > m3).astype(jnp.int32)) < _KPAD
        nlo = jnp.where(p1, lo, jnp.where(p2, m1 + 1,
                                          jnp.where(p3, m2 + 1, m3 + 1)))
        nhi = jnp.where(p1, m1, jnp.where(p2, m2, jnp.where(p3, m3, hi)))
        return (nlo, nhi)

    lo, _hi = lax.fori_loop(0, 16, bis_step,
                            (jnp.int32(0), jnp.int32(_ONE_BITS)))
    sel = (bits >= lo)
    self_f = sel.astype(jnp.float32)

    su = (lax.broadcasted_iota(jnp.int32, (128, 128), 0)
          < lax.broadcasted_iota(jnp.int32, (128, 128), 1)).astype(jnp.float32)
    within = _dot_nn(self_f, su)
    pos_ref[...] = jnp.where(sel, within.astype(jnp.int32), -1)
    grptot = within[:, 127:128] + self_f[:, 127:128]
    l40 = (lax.broadcasted_iota(jnp.int32, (_G, _G), 1)
           < lax.broadcasted_iota(jnp.int32, (_G, _G), 0)).astype(jnp.float32)
    offs_ref[...] = _dot_nn(l40, grptot)

    c_ref[...] = jnp.zeros((_M, 8), jnp.float32)
    miota = lax.broadcasted_iota(jnp.int32, (128, 1), 0)
    for g in range(_G):
        pg = pos_ref[g:g + 1, :]
        oh = (pg == miota).astype(jnp.float32)
        dg = d3_ref[g, :, :]
        off = jnp.sum(offs_ref[g:g + 1, 0:1]).astype(jnp.int32)
        offc = jnp.minimum(off, _M - 128)
        c_ref[pl.ds(offc, 128), :] = lax.dot_general(
            oh, dg, (((1,), (1,)), ((), ())),
            preferred_element_type=jnp.float32,
            precision=lax.Precision.HIGHEST)
    for k in range(_M // 128):
        ct_ref[:, k * 128:(k + 1) * 128] = jnp.transpose(
            c_ref[k * 128:(k + 1) * 128, :])

    s_lane = ct_ref[4:5, :]
    m_lane = lax.broadcasted_iota(jnp.int32, (1, _M), 1)
    rank = jnp.zeros((1, _M), jnp.int32)
    for jc in range(_M // 128):
        rs = pl.ds(jc * 128, 128)
        sj = c_ref[rs, 4:5]
        mj = jc * 128 + lax.broadcasted_iota(jnp.int32, (128, 1), 0)
        gt = (sj > s_lane) | ((sj == s_lane) & (mj < m_lane))
        rank = rank + jnp.sum(gt.astype(jnp.int32), axis=0, keepdims=True)

    ct_full = ct_ref[...]
    for kc in range(_KPAD // 128):
        kk = kc * 128 + lax.broadcasted_iota(jnp.int32, (128, 1), 0)
        oh2 = (rank == kk).astype(jnp.float32)
        lane = _dot_nt(ct_full, oh2)
        sl_ref[:, pl.ds(kc * 128, 128)] = lane
        ss_ref[pl.ds(kc * 128, 128), :] = jnp.transpose(lane)

    xj1 = sl_ref[0:1, :]
    yj1 = sl_ref[1:2, :]
    xj2 = sl_ref[2:3, :]
    yj2 = sl_ref[3:4, :]
    aj = (xj2 - xj1) * (yj2 - yj1)

    sup = jnp.zeros((1, _KPAD), jnp.float32)
    for ic in range(_KPAD // 128):
        base = ic * 128
        w = _KPAD - base
        rs = pl.ds(base, 128)
        xi1 = ss_ref[rs, 0:1]
        yi1 = ss_ref[rs, 1:2]
        xi2 = ss_ref[rs, 2:3]
        yi2 = ss_ref[rs, 3:4]
        cxj1 = xj1[:, base:]
        cyj1 = yj1[:, base:]
        cxj2 = xj2[:, base:]
        cyj2 = yj2[:, base:]
        ix = jnp.clip(jnp.minimum(xi2, cxj2) - jnp.maximum(xi1, cxj1),
                      0.0, None)
        iy = jnp.clip(jnp.minimum(yi2, cyj2) - jnp.maximum(yi1, cyj1),
                      0.0, None)
        inter = ix * iy
        ai = (xi2 - xi1) * (yi2 - yi1)
        uni = jnp.maximum(ai + aj[:, base:] - inter, 1e-6)
        rowid = base + lax.broadcasted_iota(jnp.int32, (128, 1), 0)
        colg = base + lax.broadcasted_iota(jnp.int32, (1, w), 1)
        bad = (inter > _NMS_THR * uni) & (rowid < colg)
        colpart = jnp.max(jnp.where(bad, 1.0, 0.0), axis=0, keepdims=True)
        if base:
            colpart = jnp.concatenate(
                [jnp.zeros((1, base), jnp.float32), colpart], axis=1)
        sup = jnp.maximum(sup, colpart)
    keep = 1.0 - sup

    out_ref[0, 0:4, :] = sl_ref[0:4, :]
    out_ref[0, 4:5, :] = sl_ref[4:5, :] * keep
    out_ref[0, 5:8, :] = jnp.zeros((3, _KPAD), jnp.float32)


@jax.jit
def kernel(pred_map, anchors):
    pm = pred_map.reshape(_B, _A * _NATT, 8, 128)
    an = (anchors.reshape(_A, 8, 128, 4).transpose(0, 3, 1, 2)
          .reshape(_A * 4, 8, 128))
    out = pl.pallas_call(
        _yolof_body,
        grid=(_B,),
        in_specs=[
            pl.BlockSpec((1, _A * _NATT, 8, 128), lambda b: (b, 0, 0, 0)),
            pl.BlockSpec((_A * 4, 8, 128), lambda b: (0, 0, 0)),
        ],
        out_specs=pl.BlockSpec((1, 8, _KPAD), lambda b: (b, 0, 0)),
        out_shape=jax.ShapeDtypeStruct((_B, 8, _KPAD), jnp.float32),
        scratch_shapes=[
            pltpu.VMEM((_G, 8, 128), jnp.float32),
            pltpu.VMEM((_G, 128), jnp.int32),
            pltpu.VMEM((_G, 1), jnp.float32),
            pltpu.VMEM((_M, 8), jnp.float32),
            pltpu.VMEM((8, _M), jnp.float32),
            pltpu.VMEM((8, _KPAD), jnp.float32),
            pltpu.VMEM((_KPAD, 8), jnp.float32),
        ],
    )(pm, an)
    return jnp.transpose(out, (0, 2, 1))[:, :_K, :5]


# speedup vs baseline: 6.7749x; 1.1854x over previous
import functools

import jax
import jax.numpy as jnp
from jax import lax
from jax.experimental import pallas as pl
from jax.experimental.pallas import tpu as pltpu

_B = 8
_A = 5
_NATT = 85
_HW = 1024
_N = _A * _HW
_G = _N // 128
_K = 1000
_KPAD = 1024
_M = 1152
_STRIDE = 32.0
_NMS_THR = 0.5
_ONE_BITS = 0x3F800000


def _sig(x):
    return jax.nn.sigmoid(x)


def _dot_nt(a, b):
    return lax.dot_general(a, b, (((1,), (1,)), ((), ())),
                           preferred_element_type=jnp.float32)


def _dot_nn(a, b):
    return lax.dot_general(a, b, (((1,), (0,)), ((), ())),
                           preferred_element_type=jnp.float32)


def _yolof_body(pred_ref, anch_ref, out_ref,
                d3_ref, pos_ref, offs_ref, s_ref,
                c_ref, ct_ref, sl_ref, ss_ref):
    for a in range(_A):
        pa = pred_ref[0, a * _NATT:(a + 1) * _NATT, :, :]
        an = anch_ref[a * 4:(a + 1) * 4, :, :]
        cx = _sig(pa[0]) * _STRIDE + an[0]
        cy = _sig(pa[1]) * _STRIDE + an[1]
        w = jnp.exp(jnp.clip(pa[2], -10.0, 8.0)) * an[2]
        h = jnp.exp(jnp.clip(pa[3], -10.0, 8.0)) * an[3]
        cls_max = jnp.max(pa[5:_NATT], axis=0)
        score = _sig(pa[4]) * _sig(cls_max)
        sb = lax.bitcast_convert_type(score, jnp.int32)
        gs = pl.ds(a * 8, 8)
        d3_ref[gs, 0, :] = cx - w * 0.5
        d3_ref[gs, 1, :] = cy - h * 0.5
        d3_ref[gs, 2, :] = cx + w * 0.5
        d3_ref[gs, 3, :] = cy + h * 0.5
        d3_ref[gs, 4, :] = (sb & 255).astype(jnp.float32)
        d3_ref[gs, 5, :] = ((sb >> 8) & 255).astype(jnp.float32)
        d3_ref[gs, 6, :] = ((sb >> 16) & 255).astype(jnp.float32)
        d3_ref[gs, 7, :] = (sb >> 24).astype(jnp.float32)
        s_ref[gs, :] = score

    bits = lax.bitcast_convert_type(s_ref[...], jnp.int32)

    def bis_step(_, lh):
        lo, hi = lh
        span = hi - lo
        m1 = lo + span // 4
        m2 = lo + span // 2
        m3 = lo + (span - span // 4)
        p1 = jnp.sum((bits > m1).astype(jnp.int32)) < _KPAD
        p2 = jnp.sum((bits > m2).astype(jnp.int32)) < _KPAD
        p3 = jnp.sum((bits > m3).astype(jnp.int32)) < _KPAD
        nlo = jnp.where(p1, lo, jnp.where(p2, m1 + 1,
                                          jnp.where(p3, m2 + 1, m3 + 1)))
        nhi = jnp.where(p1, m1, jnp.where(p2, m2, jnp.where(p3, m3, hi)))
        return (nlo, nhi)

    lo, _hi = lax.fori_loop(0, 16, bis_step,
                            (jnp.int32(0), jnp.int32(_ONE_BITS)))
    sel = (bits >= lo)
    self_f = sel.astype(jnp.float32)

    su = (lax.broadcasted_iota(jnp.int32, (128, 128), 0)
          < lax.broadcasted_iota(jnp.int32, (128, 128), 1)).astype(jnp.float32)
    within = _dot_nn(self_f, su)
    pos_ref[...] = jnp.where(sel, within.astype(jnp.int32), -1)
    grptot = within[:, 127:128] + self_f[:, 127:128]
    l40 = (lax.broadcasted_iota(jnp.int32, (_G, _G), 1)
           < lax.broadcasted_iota(jnp.int32, (_G, _G), 0)).astype(jnp.float32)
    offs_ref[...] = _dot_nn(l40, grptot)

    c_ref[...] = jnp.zeros((_M, 8), jnp.float32)
    miota = lax.broadcasted_iota(jnp.int32, (128, 1), 0)
    for g in range(_G):
        pg = pos_ref[g:g + 1, :]
        oh = (pg == miota).astype(jnp.float32)
        dg = d3_ref[g, :, :]
        off = jnp.sum(offs_ref[g:g + 1, 0:1]).astype(jnp.int32)
        offc = jnp.minimum(off, _M - 128)
        c_ref[pl.ds(offc, 128), :] = _dot_nt(oh, dg)
    for k in range(_M // 128):
        ct_ref[:, k * 128:(k + 1) * 128] = jnp.transpose(
            c_ref[k * 128:(k + 1) * 128, :])

    s_lane = (((ct_ref[7:8, :].astype(jnp.int32) * 256
                + ct_ref[6:7, :].astype(jnp.int32)) * 256
               + ct_ref[5:6, :].astype(jnp.int32)) * 256
              + ct_ref[4:5, :].astype(jnp.int32))
    m_lane = lax.broadcasted_iota(jnp.int32, (1, _M), 1)
    rank = jnp.zeros((1, _M), jnp.int32)
    for jc in range(_M // 128):
        sj = jnp.transpose(s_lane[:, jc * 128:(jc + 1) * 128])
        mj = jc * 128 + lax.broadcasted_iota(jnp.int32, (128, 1), 0)
        gt = (sj > s_lane) | ((sj == s_lane) & (mj < m_lane))
        rank = rank + jnp.sum(gt.astype(jnp.int32), axis=0, keepdims=True)

    ct_full = ct_ref[...]
    for kc in range(_KPAD // 128):
        kk = kc * 128 + lax.broadcasted_iota(jnp.int32, (128, 1), 0)
        oh2 = (rank == kk).astype(jnp.float32)
        lane = _dot_nt(ct_full, oh2)
        sl_ref[:, pl.ds(kc * 128, 128)] = lane
        ss_ref[pl.ds(kc * 128, 128), :] = jnp.transpose(lane)

    xj1 = sl_ref[0:1, :]
    yj1 = sl_ref[1:2, :]
    xj2 = sl_ref[2:3, :]
    yj2 = sl_ref[3:4, :]
    aj = (xj2 - xj1) * (yj2 - yj1)

    sup = jnp.zeros((1, _KPAD), jnp.float32)
    for ic in range(_KPAD // 128):
        base = ic * 128
        w = _KPAD - base
        rs = pl.ds(base, 128)
        xi1 = ss_ref[rs, 0:1]
        yi1 = ss_ref[rs, 1:2]
        xi2 = ss_ref[rs, 2:3]
        yi2 = ss_ref[rs, 3:4]
        cxj1 = xj1[:, base:]
        cyj1 = yj1[:, base:]
        cxj2 = xj2[:, base:]
        cyj2 = yj2[:, base:]
        ix = jnp.clip(jnp.minimum(xi2, cxj2) - jnp.maximum(xi1, cxj1),
                      0.0, None)
        iy = jnp.clip(jnp.minimum(yi2, cyj2) - jnp.maximum(yi1, cyj1),
                      0.0, None)
        inter = ix * iy
        ai = (xi2 - xi1) * (yi2 - yi1)
        uni = jnp.maximum(ai + aj[:, base:] - inter, 1e-6)
        rowid = base + lax.broadcasted_iota(jnp.int32, (128, 1), 0)
        colg = base + lax.broadcasted_iota(jnp.int32, (1, w), 1)
        bad = (inter > _NMS_THR * uni) & (rowid < colg)
        colpart = jnp.max(jnp.where(bad, 1.0, 0.0), axis=0, keepdims=True)
        if base:
            colpart = jnp.concatenate(
                [jnp.zeros((1, base), jnp.float32), colpart], axis=1)
        sup = jnp.maximum(sup, colpart)
    keep = 1.0 - sup

    sb_lane = (((sl_ref[7:8, :].astype(jnp.int32) * 256
                 + sl_ref[6:7, :].astype(jnp.int32)) * 256
                + sl_ref[5:6, :].astype(jnp.int32)) * 256
               + sl_ref[4:5, :].astype(jnp.int32))
    score_lane = lax.bitcast_convert_type(sb_lane, jnp.float32)
    kept = score_lane * keep
    kept_col = jnp.concatenate(
        [jnp.transpose(kept[:, k * 128:(k + 1) * 128])
         for k in range(_KPAD // 128)], axis=0)

    out_ref[0, :, 0:4] = ss_ref[0:_K, 0:4]
    out_ref[0, :, 4:5] = kept_col[0:_K, :]


@jax.jit
def kernel(pred_map, anchors):
    pm = pred_map.reshape(_B, _A * _NATT, 8, 128)
    an = (anchors.reshape(_A, 8, 128, 4).transpose(0, 3, 1, 2)
          .reshape(_A * 4, 8, 128))
    return pl.pallas_call(
        _yolof_body,
        grid=(_B,),
        in_specs=[
            pl.BlockSpec((1, _A * _NATT, 8, 128), lambda b: (b, 0, 0, 0)),
            pl.BlockSpec((_A * 4, 8, 128), lambda b: (0, 0, 0)),
        ],
        out_specs=pl.BlockSpec((1, _K, 5), lambda b: (b, 0, 0)),
        out_shape=jax.ShapeDtypeStruct((_B, _K, 5), jnp.float32),
        scratch_shapes=[
            pltpu.VMEM((_G, 8, 128), jnp.float32),
            pltpu.VMEM((_G, 128), jnp.int32),
            pltpu.VMEM((_G, 1), jnp.float32),
            pltpu.VMEM((_G, 128), jnp.float32),
            pltpu.VMEM((_M, 8), jnp.float32),
            pltpu.VMEM((8, _M), jnp.float32),
            pltpu.VMEM((8, _KPAD), jnp.float32),
            pltpu.VMEM((_KPAD, 8), jnp.float32),
        ],
    )(pm, an)


# speedup vs baseline: 7.3685x; 1.0876x over previous
import functools

import jax
import jax.numpy as jnp
from jax import lax
from jax.experimental import pallas as pl
from jax.experimental.pallas import tpu as pltpu

_B = 8
_A = 5
_NATT = 85
_HW = 1024
_N = _A * _HW
_G = _N // 128
_K = 1000
_KPAD = 1024
_M = 1152
_STRIDE = 32.0
_NMS_THR = 0.5
_ONE_BITS = 0x3F800000


def _sig(x):
    return jax.nn.sigmoid(x)


def _dot_nt(a, b):
    return lax.dot_general(a, b, (((1,), (1,)), ((), ())),
                           preferred_element_type=jnp.float32)


def _dot_nn(a, b):
    return lax.dot_general(a, b, (((1,), (0,)), ((), ())),
                           preferred_element_type=jnp.float32)


def _yolof_body(pred_ref, anch_ref, out_ref,
                d3_ref, pos_ref, offs_ref, s_ref,
                c_ref, ct_ref, sl_ref, ss_ref):
    for a in range(_A):
        pa = pred_ref[0, a * _NATT:(a + 1) * _NATT, :, :]
        an = anch_ref[a * 4:(a + 1) * 4, :, :]
        cx = _sig(pa[0]) * _STRIDE + an[0]
        cy = _sig(pa[1]) * _STRIDE + an[1]
        w = jnp.exp(jnp.clip(pa[2], -10.0, 8.0)) * an[2]
        h = jnp.exp(jnp.clip(pa[3], -10.0, 8.0)) * an[3]
        cls_max = jnp.max(pa[5:_NATT], axis=0)
        score = _sig(pa[4]) * _sig(cls_max)
        sb = lax.bitcast_convert_type(score, jnp.int32)
        gs = pl.ds(a * 8, 8)
        d3_ref[gs, 0, :] = cx - w * 0.5
        d3_ref[gs, 1, :] = cy - h * 0.5
        d3_ref[gs, 2, :] = cx + w * 0.5
        d3_ref[gs, 3, :] = cy + h * 0.5
        d3_ref[gs, 4, :] = (sb & 255).astype(jnp.float32)
        d3_ref[gs, 5, :] = ((sb >> 8) & 255).astype(jnp.float32)
        d3_ref[gs, 6, :] = ((sb >> 16) & 255).astype(jnp.float32)
        d3_ref[gs, 7, :] = (sb >> 24).astype(jnp.float32)
        s_ref[gs, :] = score

    bits = lax.bitcast_convert_type(s_ref[...], jnp.int32)

    def bis_cond(state):
        it, _lo, _hi, nsel = state
        return (nsel > _M) & (it < 16)

    def bis_step(state):
        it, lo, hi, _nsel = state
        span = hi - lo
        m1 = lo + span // 4
        m2 = lo + span // 2
        m3 = lo + (span - span // 4)
        p1 = jnp.sum((bits > m1).astype(jnp.int32)) < _KPAD
        p2 = jnp.sum((bits > m2).astype(jnp.int32)) < _KPAD
        p3 = jnp.sum((bits > m3).astype(jnp.int32)) < _KPAD
        nlo = jnp.where(p1, lo, jnp.where(p2, m1 + 1,
                                          jnp.where(p3, m2 + 1, m3 + 1)))
        nhi = jnp.where(p1, m1, jnp.where(p2, m2, jnp.where(p3, m3, hi)))
        nsel = jnp.sum((bits >= nlo).astype(jnp.int32))
        return (it + 1, nlo, nhi, nsel)

    _it, lo, _hi, _ns = lax.while_loop(
        bis_cond, bis_step,
        (jnp.int32(0), jnp.int32(0), jnp.int32(_ONE_BITS), jnp.int32(_N)))
    sel = (bits >= lo)
    self_f = sel.astype(jnp.float32)

    su = (lax.broadcasted_iota(jnp.int32, (128, 128), 0)
          < lax.broadcasted_iota(jnp.int32, (128, 128), 1)).astype(jnp.float32)
    within = _dot_nn(self_f, su)
    pos_ref[...] = jnp.where(sel, within.astype(jnp.int32), -1)
    grptot = within[:, 127:128] + self_f[:, 127:128]
    l40 = (lax.broadcasted_iota(jnp.int32, (_G, _G), 1)
           < lax.broadcasted_iota(jnp.int32, (_G, _G), 0)).astype(jnp.float32)
    offs_ref[...] = _dot_nn(l40, grptot)

    c_ref[...] = jnp.zeros((_M + 128, 8), jnp.float32)
    miota = lax.broadcasted_iota(jnp.int32, (128, 1), 0)
    for g in range(_G):
        pg = pos_ref[g:g + 1, :]
        oh = (pg == miota).astype(jnp.float32)
        dg = d3_ref[g, :, :]
        off = jnp.sum(offs_ref[g:g + 1, 0:1]).astype(jnp.int32)
        offc = jnp.minimum(off, _M)
        c_ref[pl.ds(offc, 128), :] = _dot_nt(oh, dg)
    for k in range(_M // 128):
        ct_ref[:, k * 128:(k + 1) * 128] = jnp.transpose(
            c_ref[k * 128:(k + 1) * 128, :])

    s_lane = (((ct_ref[7:8, :].astype(jnp.int32) * 256
                + ct_ref[6:7, :].astype(jnp.int32)) * 256
               + ct_ref[5:6, :].astype(jnp.int32)) * 256
              + ct_ref[4:5, :].astype(jnp.int32))
    m_lane = lax.broadcasted_iota(jnp.int32, (1, _M), 1)
    rank = jnp.zeros((1, _M), jnp.int32)
    for jc in range(_M // 128):
        sj = jnp.transpose(s_lane[:, jc * 128:(jc + 1) * 128])
        mj = jc * 128 + lax.broadcasted_iota(jnp.int32, (128, 1), 0)
        gt = (sj > s_lane) | ((sj == s_lane) & (mj < m_lane))
        rank = rank + jnp.sum(gt.astype(jnp.int32), axis=0, keepdims=True)

    ct_full = ct_ref[...]
    for kc in range(_KPAD // 128):
        kk = kc * 128 + lax.broadcasted_iota(jnp.int32, (128, 1), 0)
        oh2 = (rank == kk).astype(jnp.float32)
        lane = _dot_nt(ct_full, oh2)
        sl_ref[:, pl.ds(kc * 128, 128)] = lane
        ss_ref[pl.ds(kc * 128, 128), :] = jnp.transpose(lane)

    xj1 = sl_ref[0:1, :]
    yj1 = sl_ref[1:2, :]
    xj2 = sl_ref[2:3, :]
    yj2 = sl_ref[3:4, :]
    aj = (xj2 - xj1) * (yj2 - yj1)

    sup = jnp.zeros((1, _KPAD), jnp.float32)
    for ic in range(_KPAD // 128):
        base = ic * 128
        w = _KPAD - base
        rs = pl.ds(base, 128)
        xi1 = ss_ref[rs, 0:1]
        yi1 = ss_ref[rs, 1:2]
        xi2 = ss_ref[rs, 2:3]
        yi2 = ss_ref[rs, 3:4]
        cxj1 = xj1[:, base:]
        cyj1 = yj1[:, base:]
        cxj2 = xj2[:, base:]
        cyj2 = yj2[:, base:]
        ix = jnp.clip(jnp.minimum(xi2, cxj2) - jnp.maximum(xi1, cxj1),
                      0.0, None)
        iy = jnp.clip(jnp.minimum(yi2, cyj2) - jnp.maximum(yi1, cyj1),
                      0.0, None)
        inter = ix * iy
        ai = (xi2 - xi1) * (yi2 - yi1)
        uni = jnp.maximum(ai + aj[:, base:] - inter, 1e-6)
        rowid = base + lax.broadcasted_iota(jnp.int32, (128, 1), 0)
        colg = base + lax.broadcasted_iota(jnp.int32, (1, w), 1)
        bad = (inter > _NMS_THR * uni) & (rowid < colg)
        colpart = jnp.max(jnp.where(bad, 1.0, 0.0), axis=0, keepdims=True)
        if base:
            colpart = jnp.concatenate(
                [jnp.zeros((1, base), jnp.float32), colpart], axis=1)
        sup = jnp.maximum(sup, colpart)
    keep = 1.0 - sup

    sb_lane = (((sl_ref[7:8, :].astype(jnp.int32) * 256
                 + sl_ref[6:7, :].astype(jnp.int32)) * 256
                + sl_ref[5:6, :].astype(jnp.int32)) * 256
               + sl_ref[4:5, :].astype(jnp.int32))
    score_lane = lax.bitcast_convert_type(sb_lane, jnp.float32)
    kept = score_lane * keep
    kept_col = jnp.concatenate(
        [jnp.transpose(kept[:, k * 128:(k + 1) * 128])
         for k in range(_KPAD // 128)], axis=0)

    out_ref[0, :, 0:4] = ss_ref[0:_K, 0:4]
    out_ref[0, :, 4:5] = kept_col[0:_K, :]


@jax.jit
def kernel(pred_map, anchors):
    pm = pred_map.reshape(_B, _A * _NATT, 8, 128)
    an = (anchors.reshape(_A, 8, 128, 4).transpose(0, 3, 1, 2)
          .reshape(_A * 4, 8, 128))
    return pl.pallas_call(
        _yolof_body,
        grid=(_B,),
        in_specs=[
            pl.BlockSpec((1, _A * _NATT, 8, 128), lambda b: (b, 0, 0, 0)),
            pl.BlockSpec((_A * 4, 8, 128), lambda b: (0, 0, 0)),
        ],
        out_specs=pl.BlockSpec((1, _K, 5), lambda b: (b, 0, 0)),
        out_shape=jax.ShapeDtypeStruct((_B, _K, 5), jnp.float32),
        scratch_shapes=[
            pltpu.VMEM((_G, 8, 128), jnp.float32),
            pltpu.VMEM((_G, 128), jnp.int32),
            pltpu.VMEM((_G, 1), jnp.float32),
            pltpu.VMEM((_G, 128), jnp.float32),
            pltpu.VMEM((_M + 128, 8), jnp.float32),
            pltpu.VMEM((8, _M), jnp.float32),
            pltpu.VMEM((8, _KPAD), jnp.float32),
            pltpu.VMEM((_KPAD, 8), jnp.float32),
        ],
    )(pm, an)
